# Initial kernel scaffold; baseline (speedup 1.0000x reference)
import jax
import jax.numpy as jnp
from jax.experimental import pallas as pl


def kernel(z, target, codebook, E):
    raise NotImplementedError("write your pallas kernel here")


# speedup vs baseline: 1.3426x; 1.3426x over previous
import functools

import jax
import jax.numpy as jnp
from jax import lax
from jax.experimental import pallas as pl
from jax.experimental.pallas import tpu as pltpu
from jax.experimental.pallas import tpu_sc as plsc

_K = 1024
_D = 256
_BN = 16 * 576


_TM = 512


def _argmax_body(z_ref, cb_ref, idx_ref):
    c = cb_ref[...]
    inv_norm = 1.0 / jnp.maximum(jnp.sqrt(jnp.sum(c * c, axis=1)), 1e-12)
    logits = lax.dot_general(
        z_ref[...], c, (((1,), (1,)), ((), ())),
        preferred_element_type=jnp.float32,
    )
    scaled = logits * inv_norm[None, :]
    idx_ref[...] = jnp.argmax(scaled, axis=1).astype(jnp.int32)


def _compute_indices(z2d, codebook):
    grid = _BN // _TM
    return pl.pallas_call(
        _argmax_body,
        grid=(grid,),
        in_specs=[
            pl.BlockSpec((_TM, _D), lambda i: (i, 0)),
            pl.BlockSpec((_K, _D), lambda i: (0, 0)),
        ],
        out_specs=pl.BlockSpec((_TM,), lambda i: (i,)),
        out_shape=jax.ShapeDtypeStruct((_BN,), jnp.int32),
    )(z2d, codebook)


_NC, _NS, _L = 2, 16, 16
_NW = _NC * _NS
_BPW = _BN // _NW
_CH = 96
_NCH = _BPW // _CH


def _sc_gather_mul(idx3d, target2d, E):
    mesh = plsc.VectorSubcoreMesh(core_axis_name="c", subcore_axis_name="s")

    @functools.partial(
        pl.kernel,
        mesh=mesh,
        out_type=jax.ShapeDtypeStruct((_BN, _D), jnp.float32),
        scratch_types=[
            pltpu.VMEM((_NCH, _CH), jnp.int32),
            pltpu.VMEM((_BPW, _D), jnp.float32),
            pltpu.VMEM((_CH, _D), jnp.float32),
            pltpu.SemaphoreType.DMA,
        ],
    )
    def body(idx_hbm, tgt_hbm, e_hbm, out_hbm, idx_v, rows_v, tbuf, sem):
        wid = lax.axis_index("s") * _NC + lax.axis_index("c")
        base = wid * _BPW
        pltpu.sync_copy(idx_hbm.at[wid], idx_v)
        for c in range(_NCH):
            pltpu.async_copy(
                e_hbm.at[idx_v.at[c]], rows_v.at[pl.ds(c * _CH, _CH)], sem
            ).wait()
            pltpu.sync_copy(tgt_hbm.at[pl.ds(base + c * _CH, _CH)], tbuf)

            def row_body(r, _, c=c):
                for l in range(_D // _L):
                    sl = pl.ds(l * _L, _L)
                    tbuf[r, sl] = tbuf[r, sl] * (rows_v[c * _CH + r, sl] + 1.0)
                return 0

            lax.fori_loop(0, _CH, row_body, 0)
            pltpu.sync_copy(tbuf, out_hbm.at[pl.ds(base + c * _CH, _CH)])

    return body(idx3d, target2d, E)


def kernel(z, target, codebook, E):
    B, N, D = z.shape
    z2d = z.reshape(B * N, D)
    idx = _compute_indices(z2d, codebook)
    idx3d = idx.reshape(_NW, _NCH, _CH)
    out2d = _sc_gather_mul(idx3d, target.reshape(B * N, D), E)
    return out2d.reshape(B, N, D)
